# Initial kernel scaffold; baseline (speedup 1.0000x reference)
import jax
import jax.numpy as jnp
from jax.experimental import pallas as pl


def kernel(x, x_sem, edge_index, edge_attr, batch, label_emb, type_emb, W_sem, b_sem, role_emb, child_emb, W1_0, b1_0, W2_0, b2_0, W1_1, b1_1, W2_1, b2_1, W_proj, b_proj):
    raise NotImplementedError("write your pallas kernel here")


# speedup vs baseline: 2.0136x; 2.0136x over previous
import functools

import jax
import jax.numpy as jnp
from jax import lax
from jax.experimental import pallas as pl
from jax.experimental.pallas import tpu as pltpu
from jax.experimental.pallas import tpu_sc as plsc

N = 10000
E = 320000
HID = 128
B = 64
NCHILD = 200

NC = 2
NS = 16
NW = NC * NS

CH = 80
EPS = E // NS
NCHUNK = EPS // CH
BLK = 50
NBLK = NCHUNK // BLK
NROWCH = N // CH

NHALF = N // NC
ACC_ROWS = 5008
NFULL = NHALF // CH
TAIL = NHALF - NFULL * CH
ZTAIL = ACC_ROWS - NFULL * CH

MB = 400
NB = N // MB

_mesh = plsc.VectorSubcoreMesh(core_axis_name="c", subcore_axis_name="s")
_PREC = lax.Precision.HIGHEST


@functools.partial(
    pl.kernel,
    out_type=jax.ShapeDtypeStruct((N, HID), jnp.float32),
    mesh=_mesh,
    scratch_types=[
        pltpu.VMEM((CH,), jnp.int32),
        pltpu.VMEM((CH,), jnp.int32),
        pltpu.VMEM((CH, HID), jnp.float32),
        pltpu.SemaphoreType.DMA,
    ],
)
def _node_emb(x0_hbm, x1_hbm, lab_hbm, typ_hbm, out_hbm, xi0, xi1, rows, sem):
    w = lax.axis_index("s") * NC + lax.axis_index("c")
    for k in range((NROWCH + NW - 1) // NW):
        c = w + k * NW

        @pl.when(c < NROWCH)
        def _():
            pltpu.sync_copy(x0_hbm.at[c, 0], xi0)
            pltpu.sync_copy(x1_hbm.at[c, 0], xi1)
            pltpu.async_copy(lab_hbm.at[xi0], rows, sem).wait()
            pltpu.async_copy(typ_hbm.at[xi1], rows, sem, add=True).wait()
            pltpu.sync_copy(rows, out_hbm.at[pl.ds(c * CH, CH)])


@functools.partial(
    pl.kernel,
    out_type=jax.ShapeDtypeStruct((N, HID), jnp.float32),
    mesh=_mesh,
    scratch_types=[
        pltpu.VMEM((BLK, CH), jnp.int32),
        pltpu.VMEM((BLK, CH), jnp.int32),
        pltpu.VMEM((BLK, CH), jnp.int32),
        pltpu.VMEM((BLK, CH), jnp.int32),
        pltpu.VMEM((CH, HID), jnp.float32),
        pltpu.VMEM_SHARED((ACC_ROWS, HID), jnp.float32),
        pltpu.SemaphoreType.DMA,
    ],
)
def _msg_pass(src_hbm, dst0_hbm, dst1_hbm, role_hbm, child_hbm,
              h_hbm, remb_hbm, cemb_hbm,
              out_hbm, si, di, ri, ci, m, acc, sem):
    cid = lax.axis_index("c")
    sid = lax.axis_index("s")

    zv = jnp.zeros((16,), jnp.float32)

    def zrow(r, _):
        for cc in range(HID // 16):
            m[r, pl.ds(cc * 16, 16)] = zv
        return 0

    lax.fori_loop(0, CH, zrow, 0)

    for k in range((NFULL + NS - 1) // NS):
        c = sid + k * NS

        @pl.when(c < NFULL)
        def _():
            pltpu.sync_copy(m, acc.at[pl.ds(c * CH, CH)])

    @pl.when(sid == NS - 1)
    def _():
        pltpu.sync_copy(m.at[pl.ds(0, ZTAIL)],
                        acc.at[pl.ds(NFULL * CH, ZTAIL)])

    plsc.subcore_barrier()

    def block(bj, _):
        pltpu.sync_copy(src_hbm.at[sid, bj], si)
        pltpu.sync_copy(role_hbm.at[sid, bj], ri)
        pltpu.sync_copy(child_hbm.at[sid, bj], ci)

        @pl.when(cid == 0)
        def _():
            pltpu.sync_copy(dst0_hbm.at[sid, bj], di)

        @pl.when(cid == 1)
        def _():
            pltpu.sync_copy(dst1_hbm.at[sid, bj], di)

        def chunk(j, _):
            pltpu.async_copy(h_hbm.at[si.at[j]], m, sem).wait()
            pltpu.async_copy(remb_hbm.at[ri.at[j]], m, sem, add=True).wait()
            pltpu.async_copy(cemb_hbm.at[ci.at[j]], m, sem, add=True).wait()

            def rrow(r, _):
                for cc in range(HID // 16):
                    sl = pl.ds(cc * 16, 16)
                    m[r, sl] = jnp.maximum(m[r, sl], 0.0)
                return 0

            lax.fori_loop(0, CH, rrow, 0)
            pltpu.sync_copy(m, acc.at[di.at[j]], add=True)
            return 0

        lax.fori_loop(0, BLK, chunk, 0)
        return 0

    lax.fori_loop(0, NBLK, block, 0)
    plsc.subcore_barrier()

    for k in range((NFULL + NS - 1) // NS):
        c = sid + k * NS

        @pl.when(c < NFULL)
        def _():
            pltpu.sync_copy(acc.at[pl.ds(c * CH, CH)],
                            out_hbm.at[pl.ds(cid * NHALF + c * CH, CH)])

    @pl.when(sid == NS - 1)
    def _():
        pltpu.sync_copy(acc.at[pl.ds(NFULL * CH, TAIL)],
                        out_hbm.at[pl.ds(cid * NHALF + NFULL * CH, TAIL)])


def _sem_body(base_ref, xs_ref, w_ref, b_ref, o_ref):
    o_ref[...] = (base_ref[...] +
                  jnp.dot(xs_ref[...], w_ref[...],
                          preferred_element_type=jnp.float32, precision=_PREC) +
                  b_ref[...])


def _sem_matmul(base, x_sem, W_sem, b_sem2d):
    k = x_sem.shape[1]
    return pl.pallas_call(
        _sem_body,
        grid=(NB,),
        in_specs=[
            pl.BlockSpec((MB, HID), lambda i: (i, 0)),
            pl.BlockSpec((MB, k), lambda i: (i, 0)),
            pl.BlockSpec((k, HID), lambda i: (0, 0)),
            pl.BlockSpec((1, HID), lambda i: (0, 0)),
        ],
        out_specs=pl.BlockSpec((MB, HID), lambda i: (i, 0)),
        out_shape=jax.ShapeDtypeStruct((N, HID), jnp.float32),
    )(base, x_sem, W_sem, b_sem2d)


def _mlp_body(h_ref, agg_ref, w1_ref, b1_ref, w2_ref, b2_ref, o_ref):
    z = h_ref[...] + agg_ref[...]
    t = jnp.maximum(
        jnp.dot(z, w1_ref[...], preferred_element_type=jnp.float32,
                precision=_PREC) + b1_ref[...], 0.0)
    o_ref[...] = (jnp.dot(t, w2_ref[...], preferred_element_type=jnp.float32,
                          precision=_PREC) + b2_ref[...])


def _mlp(h, agg, W1, b1_2d, W2, b2_2d):
    return pl.pallas_call(
        _mlp_body,
        grid=(NB,),
        in_specs=[
            pl.BlockSpec((MB, HID), lambda i: (i, 0)),
            pl.BlockSpec((MB, HID), lambda i: (i, 0)),
            pl.BlockSpec((HID, HID), lambda i: (0, 0)),
            pl.BlockSpec((1, HID), lambda i: (0, 0)),
            pl.BlockSpec((HID, HID), lambda i: (0, 0)),
            pl.BlockSpec((1, HID), lambda i: (0, 0)),
        ],
        out_specs=pl.BlockSpec((MB, HID), lambda i: (i, 0)),
        out_shape=jax.ShapeDtypeStruct((N, HID), jnp.float32),
    )(h, agg, W1, b1_2d, W2, b2_2d)


def _pool_body(h_ref, batch_ref, wp_ref, bp_ref, o_ref, sums, cnts):
    i = pl.program_id(0)

    @pl.when(i == 0)
    def _():
        sums[...] = jnp.zeros_like(sums)
        cnts[...] = jnp.zeros_like(cnts)

    b = batch_ref[0, 0]
    mask = (b[:, None] ==
            lax.broadcasted_iota(jnp.int32, (1, B), 1)).astype(jnp.float32)
    sums[...] += lax.dot_general(mask, h_ref[...], (((0,), (0,)), ((), ())),
                                 preferred_element_type=jnp.float32,
                                 precision=_PREC)
    ones = jnp.ones((MB, HID), jnp.float32)
    cnts[...] += lax.dot_general(mask, ones, (((0,), (0,)), ((), ())),
                                 preferred_element_type=jnp.float32,
                                 precision=_PREC)

    @pl.when(i == NB - 1)
    def _():
        pooled = sums[...] / jnp.maximum(cnts[...], 1.0)
        o_ref[...] = (jnp.dot(pooled, wp_ref[...],
                              preferred_element_type=jnp.float32,
                              precision=_PREC) + bp_ref[...])


def _pool_proj(h, batch3d, W_proj, b_proj2d):
    return pl.pallas_call(
        _pool_body,
        grid=(NB,),
        in_specs=[
            pl.BlockSpec((MB, HID), lambda i: (i, 0)),
            pl.BlockSpec((1, 1, MB), lambda i: (i, 0, 0)),
            pl.BlockSpec((HID, HID), lambda i: (0, 0)),
            pl.BlockSpec((1, HID), lambda i: (0, 0)),
        ],
        out_specs=pl.BlockSpec((B, HID), lambda i: (0, 0)),
        out_shape=jax.ShapeDtypeStruct((B, HID), jnp.float32),
        scratch_shapes=[
            pltpu.VMEM((B, HID), jnp.float32),
            pltpu.VMEM((B, HID), jnp.float32),
        ],
    )(h, batch3d, W_proj, b_proj2d)


def kernel(x, x_sem, edge_index, edge_attr, batch, label_emb, type_emb,
           W_sem, b_sem, role_emb, child_emb,
           W1_0, b1_0, W2_0, b2_0, W1_1, b1_1, W2_1, b2_1,
           W_proj, b_proj):
    xt = x.T.astype(jnp.int32)
    x0 = xt[0].reshape(NROWCH, 1, CH)
    x1 = xt[1].reshape(NROWCH, 1, CH)
    ei = edge_index.astype(jnp.int32)
    src2d = ei[0].reshape(NS, NBLK, BLK, CH)
    dst = ei[1]
    dst0 = jnp.minimum(dst, NHALF).reshape(NS, NBLK, BLK, CH)
    dst1 = jnp.where(dst >= NHALF, dst - NHALF,
                     NHALF).reshape(NS, NBLK, BLK, CH)
    ea = edge_attr.T.astype(jnp.int32)
    role2d = ea[0].reshape(NS, NBLK, BLK, CH)
    child2d = jnp.clip(ea[1], 0, NCHILD - 1).reshape(NS, NBLK, BLK, CH)
    batch3d = batch.astype(jnp.int32).reshape(NB, 1, MB)

    base = _node_emb(x0, x1, label_emb, type_emb)
    h = _sem_matmul(base, x_sem, W_sem, b_sem.reshape(1, HID))
    for (W1, b1, W2, b2) in ((W1_0, b1_0, W2_0, b2_0),
                             (W1_1, b1_1, W2_1, b2_1)):
        agg = _msg_pass(src2d, dst0, dst1, role2d, child2d, h, role_emb,
                        child_emb)
        h = _mlp(h, agg, W1, b1.reshape(1, HID), W2, b2.reshape(1, HID))
    return _pool_proj(h, batch3d, W_proj, b_proj.reshape(1, HID))


# speedup vs baseline: 3.2989x; 1.6383x over previous
import functools

import jax
import jax.numpy as jnp
from jax import lax
from jax.experimental import pallas as pl
from jax.experimental.pallas import tpu as pltpu
from jax.experimental.pallas import tpu_sc as plsc

N = 10000
E = 320000
HID = 128
B = 64
NROLE = 200
NCHILD = 200

NC = 2
NS = 16
NW = NC * NS

CH = 80
EPS = E // NS
NCHUNK = EPS // CH
BLK = 50
NBLK = NCHUNK // BLK
NPAIR = BLK // 2
NROWCH = N // CH
NRC = NROLE * NCHILD

NHALF = N // NC
ACC_ROWS = 5008
NFULL = NHALF // CH
TAIL = NHALF - NFULL * CH
ZTAIL = ACC_ROWS - NFULL * CH

MB = 400
NB = N // MB

_mesh = plsc.VectorSubcoreMesh(core_axis_name="c", subcore_axis_name="s")
_PREC = lax.Precision.HIGHEST


@functools.partial(
    pl.kernel,
    out_type=jax.ShapeDtypeStruct((N, HID), jnp.float32),
    mesh=_mesh,
    scratch_types=[
        pltpu.VMEM((CH,), jnp.int32),
        pltpu.VMEM((CH,), jnp.int32),
        pltpu.VMEM((CH, HID), jnp.float32),
        pltpu.SemaphoreType.DMA,
    ],
)
def _node_emb(x0_hbm, x1_hbm, lab_hbm, typ_hbm, out_hbm, xi0, xi1, rows, sem):
    w = lax.axis_index("s") * NC + lax.axis_index("c")
    for k in range((NROWCH + NW - 1) // NW):
        c = w + k * NW

        @pl.when(c < NROWCH)
        def _():
            pltpu.sync_copy(x0_hbm.at[c, 0], xi0)
            pltpu.sync_copy(x1_hbm.at[c, 0], xi1)
            pltpu.async_copy(lab_hbm.at[xi0], rows, sem).wait()
            pltpu.async_copy(typ_hbm.at[xi1], rows, sem, add=True).wait()
            pltpu.sync_copy(rows, out_hbm.at[pl.ds(c * CH, CH)])


@functools.partial(
    pl.kernel,
    out_type=jax.ShapeDtypeStruct((N, HID), jnp.float32),
    mesh=_mesh,
    scratch_types=[
        pltpu.VMEM((BLK, CH), jnp.int32),
        pltpu.VMEM((BLK, CH), jnp.int32),
        pltpu.VMEM((BLK, CH), jnp.int32),
        pltpu.VMEM((CH, HID), jnp.float32),
        pltpu.VMEM((CH, HID), jnp.float32),
        pltpu.VMEM((CH, HID), jnp.float32),
        pltpu.VMEM((CH, HID), jnp.float32),
        pltpu.VMEM_SHARED((ACC_ROWS, HID), jnp.float32),
        pltpu.SemaphoreType.DMA,
        pltpu.SemaphoreType.DMA,
        pltpu.SemaphoreType.DMA,
        pltpu.SemaphoreType.DMA,
    ],
)
def _msg_pass(src_hbm, dst0_hbm, dst1_hbm, rc_hbm,
              h_hbm, pair_hbm,
              out_hbm, si, di, rci, ma, aa, mb, ab, acc,
              sha, spa, shb, spb):
    cid = lax.axis_index("c")
    sid = lax.axis_index("s")

    zv = jnp.zeros((16,), jnp.float32)

    def zrow(r, _):
        for cc in range(HID // 16):
            ma[r, pl.ds(cc * 16, 16)] = zv
        return 0

    lax.fori_loop(0, CH, zrow, 0)

    for k in range((NFULL + NS - 1) // NS):
        c = sid + k * NS

        @pl.when(c < NFULL)
        def _():
            pltpu.sync_copy(ma, acc.at[pl.ds(c * CH, CH)])

    @pl.when(sid == NS - 1)
    def _():
        pltpu.sync_copy(ma.at[pl.ds(0, ZTAIL)],
                        acc.at[pl.ds(NFULL * CH, ZTAIL)])

    plsc.subcore_barrier()

    def relu_add(m, a):
        def rrow(r, _):
            for cc in range(HID // 16):
                sl = pl.ds(cc * 16, 16)
                m[r, sl] = jnp.maximum(m[r, sl] + a[r, sl], 0.0)
            return 0

        lax.fori_loop(0, CH, rrow, 0)

    def block(bj, _):
        pltpu.sync_copy(src_hbm.at[sid, bj], si)
        pltpu.sync_copy(rc_hbm.at[sid, bj], rci)

        @pl.when(cid == 0)
        def _():
            pltpu.sync_copy(dst0_hbm.at[sid, bj], di)

        @pl.when(cid == 1)
        def _():
            pltpu.sync_copy(dst1_hbm.at[sid, bj], di)

        def pair(k, _):
            j0 = 2 * k
            j1 = j0 + 1
            ha = pltpu.async_copy(h_hbm.at[si.at[j0]], ma, sha)
            pa = pltpu.async_copy(pair_hbm.at[rci.at[j0]], aa, spa)
            hb = pltpu.async_copy(h_hbm.at[si.at[j1]], mb, shb)
            pb = pltpu.async_copy(pair_hbm.at[rci.at[j1]], ab, spb)
            ha.wait()
            pa.wait()
            relu_add(ma, aa)
            pltpu.sync_copy(ma, acc.at[di.at[j0]], add=True)
            hb.wait()
            pb.wait()
            relu_add(mb, ab)
            pltpu.sync_copy(mb, acc.at[di.at[j1]], add=True)
            return 0

        lax.fori_loop(0, NPAIR, pair, 0)
        return 0

    lax.fori_loop(0, NBLK, block, 0)
    plsc.subcore_barrier()

    for k in range((NFULL + NS - 1) // NS):
        c = sid + k * NS

        @pl.when(c < NFULL)
        def _():
            pltpu.sync_copy(acc.at[pl.ds(c * CH, CH)],
                            out_hbm.at[pl.ds(cid * NHALF + c * CH, CH)])

    @pl.when(sid == NS - 1)
    def _():
        pltpu.sync_copy(acc.at[pl.ds(NFULL * CH, TAIL)],
                        out_hbm.at[pl.ds(cid * NHALF + NFULL * CH, TAIL)])


def _pair_body(role_ref, child_ref, o_ref):
    for rr in range(8):
        o_ref[pl.ds(rr * NCHILD, NCHILD), :] = (
            child_ref[...] + role_ref[pl.ds(rr, 1), :])


def _pair_table(role_emb, child_emb):
    return pl.pallas_call(
        _pair_body,
        grid=(NROLE // 8,),
        in_specs=[
            pl.BlockSpec((8, HID), lambda i: (i, 0)),
            pl.BlockSpec((NCHILD, HID), lambda i: (0, 0)),
        ],
        out_specs=pl.BlockSpec((8 * NCHILD, HID), lambda i: (i, 0)),
        out_shape=jax.ShapeDtypeStruct((NRC, HID), jnp.float32),
    )(role_emb, child_emb)


def _sem_body(base_ref, xs_ref, w_ref, b_ref, o_ref):
    o_ref[...] = (base_ref[...] +
                  jnp.dot(xs_ref[...], w_ref[...],
                          preferred_element_type=jnp.float32, precision=_PREC) +
                  b_ref[...])


def _sem_matmul(base, x_sem, W_sem, b_sem2d):
    k = x_sem.shape[1]
    return pl.pallas_call(
        _sem_body,
        grid=(NB,),
        in_specs=[
            pl.BlockSpec((MB, HID), lambda i: (i, 0)),
            pl.BlockSpec((MB, k), lambda i: (i, 0)),
            pl.BlockSpec((k, HID), lambda i: (0, 0)),
            pl.BlockSpec((1, HID), lambda i: (0, 0)),
        ],
        out_specs=pl.BlockSpec((MB, HID), lambda i: (i, 0)),
        out_shape=jax.ShapeDtypeStruct((N, HID), jnp.float32),
    )(base, x_sem, W_sem, b_sem2d)


def _mlp_body(h_ref, agg_ref, w1_ref, b1_ref, w2_ref, b2_ref, o_ref):
    z = h_ref[...] + agg_ref[...]
    t = jnp.maximum(
        jnp.dot(z, w1_ref[...], preferred_element_type=jnp.float32,
                precision=_PREC) + b1_ref[...], 0.0)
    o_ref[...] = (jnp.dot(t, w2_ref[...], preferred_element_type=jnp.float32,
                          precision=_PREC) + b2_ref[...])


def _mlp(h, agg, W1, b1_2d, W2, b2_2d):
    return pl.pallas_call(
        _mlp_body,
        grid=(NB,),
        in_specs=[
            pl.BlockSpec((MB, HID), lambda i: (i, 0)),
            pl.BlockSpec((MB, HID), lambda i: (i, 0)),
            pl.BlockSpec((HID, HID), lambda i: (0, 0)),
            pl.BlockSpec((1, HID), lambda i: (0, 0)),
            pl.BlockSpec((HID, HID), lambda i: (0, 0)),
            pl.BlockSpec((1, HID), lambda i: (0, 0)),
        ],
        out_specs=pl.BlockSpec((MB, HID), lambda i: (i, 0)),
        out_shape=jax.ShapeDtypeStruct((N, HID), jnp.float32),
    )(h, agg, W1, b1_2d, W2, b2_2d)


def _pool_body(h_ref, batch_ref, wp_ref, bp_ref, o_ref, sums, cnts):
    i = pl.program_id(0)

    @pl.when(i == 0)
    def _():
        sums[...] = jnp.zeros_like(sums)
        cnts[...] = jnp.zeros_like(cnts)

    b = batch_ref[0, 0]
    mask = (b[:, None] ==
            lax.broadcasted_iota(jnp.int32, (1, B), 1)).astype(jnp.float32)
    sums[...] += lax.dot_general(mask, h_ref[...], (((0,), (0,)), ((), ())),
                                 preferred_element_type=jnp.float32,
                                 precision=_PREC)
    ones = jnp.ones((MB, HID), jnp.float32)
    cnts[...] += lax.dot_general(mask, ones, (((0,), (0,)), ((), ())),
                                 preferred_element_type=jnp.float32,
                                 precision=_PREC)

    @pl.when(i == NB - 1)
    def _():
        pooled = sums[...] / jnp.maximum(cnts[...], 1.0)
        o_ref[...] = (jnp.dot(pooled, wp_ref[...],
                              preferred_element_type=jnp.float32,
                              precision=_PREC) + bp_ref[...])


def _pool_proj(h, batch3d, W_proj, b_proj2d):
    return pl.pallas_call(
        _pool_body,
        grid=(NB,),
        in_specs=[
            pl.BlockSpec((MB, HID), lambda i: (i, 0)),
            pl.BlockSpec((1, 1, MB), lambda i: (i, 0, 0)),
            pl.BlockSpec((HID, HID), lambda i: (0, 0)),
            pl.BlockSpec((1, HID), lambda i: (0, 0)),
        ],
        out_specs=pl.BlockSpec((B, HID), lambda i: (0, 0)),
        out_shape=jax.ShapeDtypeStruct((B, HID), jnp.float32),
        scratch_shapes=[
            pltpu.VMEM((B, HID), jnp.float32),
            pltpu.VMEM((B, HID), jnp.float32),
        ],
    )(h, batch3d, W_proj, b_proj2d)


def kernel(x, x_sem, edge_index, edge_attr, batch, label_emb, type_emb,
           W_sem, b_sem, role_emb, child_emb,
           W1_0, b1_0, W2_0, b2_0, W1_1, b1_1, W2_1, b2_1,
           W_proj, b_proj):
    xt = x.T.astype(jnp.int32)
    x0 = xt[0].reshape(NROWCH, 1, CH)
    x1 = xt[1].reshape(NROWCH, 1, CH)
    ei = edge_index.astype(jnp.int32)
    src2d = ei[0].reshape(NS, NBLK, BLK, CH)
    dst = ei[1]
    dst0 = jnp.minimum(dst, NHALF).reshape(NS, NBLK, BLK, CH)
    dst1 = jnp.where(dst >= NHALF, dst - NHALF,
                     NHALF).reshape(NS, NBLK, BLK, CH)
    ea = edge_attr.T.astype(jnp.int32)
    rc2d = (ea[0] * NCHILD +
            jnp.clip(ea[1], 0, NCHILD - 1)).reshape(NS, NBLK, BLK, CH)
    batch3d = batch.astype(jnp.int32).reshape(NB, 1, MB)

    base = _node_emb(x0, x1, label_emb, type_emb)
    pair = _pair_table(role_emb, child_emb)
    h = _sem_matmul(base, x_sem, W_sem, b_sem.reshape(1, HID))
    for (W1, b1, W2, b2) in ((W1_0, b1_0, W2_0, b2_0),
                             (W1_1, b1_1, W2_1, b2_1)):
        agg = _msg_pass(src2d, dst0, dst1, rc2d, h, pair)
        h = _mlp(h, agg, W1, b1.reshape(1, HID), W2, b2.reshape(1, HID))
    return _pool_proj(h, batch3d, W_proj, b_proj.reshape(1, HID))


# speedup vs baseline: 4.5179x; 1.3695x over previous
import functools

import jax
import jax.numpy as jnp
from jax import lax
from jax.experimental import pallas as pl
from jax.experimental.pallas import tpu as pltpu
from jax.experimental.pallas import tpu_sc as plsc

N = 10000
E = 320000
HID = 128
B = 64
NROLE = 200
NCHILD = 200

NC = 2
NS = 16
NW = NC * NS

CH = 80
EPS = E // NS
NCHUNK = EPS // CH
BLK = 50
NBLK = NCHUNK // BLK
NPAIR = BLK // 2
NROWCH = N // CH
NRC = NROLE * NCHILD

NHALF = N // NC
ACC_ROWS = 5008
NFULL = NHALF // CH
TAIL = NHALF - NFULL * CH
ZTAIL = ACC_ROWS - NFULL * CH

MB = 400
NB = N // MB

_mesh = plsc.VectorSubcoreMesh(core_axis_name="c", subcore_axis_name="s")
_PREC = lax.Precision.HIGHEST


@functools.partial(
    pl.kernel,
    out_type=jax.ShapeDtypeStruct((N, HID), jnp.float32),
    mesh=_mesh,
    scratch_types=[
        pltpu.VMEM((CH,), jnp.int32),
        pltpu.VMEM((CH,), jnp.int32),
        pltpu.VMEM((CH, HID), jnp.float32),
        pltpu.SemaphoreType.DMA,
    ],
)
def _node_emb(x0_hbm, x1_hbm, lab_hbm, typ_hbm, out_hbm, xi0, xi1, rows, sem):
    w = lax.axis_index("s") * NC + lax.axis_index("c")
    for k in range((NROWCH + NW - 1) // NW):
        c = w + k * NW

        @pl.when(c < NROWCH)
        def _():
            pltpu.sync_copy(x0_hbm.at[c, 0], xi0)
            pltpu.sync_copy(x1_hbm.at[c, 0], xi1)
            pltpu.async_copy(lab_hbm.at[xi0], rows, sem).wait()
            pltpu.async_copy(typ_hbm.at[xi1], rows, sem, add=True).wait()
            pltpu.sync_copy(rows, out_hbm.at[pl.ds(c * CH, CH)])


@functools.partial(
    pl.kernel,
    out_type=jax.ShapeDtypeStruct((N, HID), jnp.float32),
    mesh=_mesh,
    scratch_types=[
        pltpu.VMEM((BLK, CH), jnp.int32),
        pltpu.VMEM((BLK, CH), jnp.int32),
        pltpu.VMEM((BLK, CH), jnp.int32),
        pltpu.VMEM((CH, HID), jnp.float32),
        pltpu.VMEM((CH, HID), jnp.float32),
        pltpu.VMEM((CH, HID), jnp.float32),
        pltpu.VMEM((CH, HID), jnp.float32),
        pltpu.VMEM_SHARED((ACC_ROWS, HID), jnp.float32),
        pltpu.SemaphoreType.DMA,
        pltpu.SemaphoreType.DMA,
        pltpu.SemaphoreType.DMA,
        pltpu.SemaphoreType.DMA,
    ],
)
def _msg_pass(src_hbm, dst0_hbm, dst1_hbm, rc_hbm,
              h_hbm, pair_hbm,
              out_hbm, si, di, rci, ma, aa, mb, ab, acc,
              sha, spa, shb, spb):
    cid = lax.axis_index("c")
    sid = lax.axis_index("s")

    zv = jnp.zeros((16,), jnp.float32)

    def zrow(r, _):
        for cc in range(HID // 16):
            ma[r, pl.ds(cc * 16, 16)] = zv
        return 0

    lax.fori_loop(0, CH, zrow, 0)

    for k in range((NFULL + NS - 1) // NS):
        c = sid + k * NS

        @pl.when(c < NFULL)
        def _():
            pltpu.sync_copy(ma, acc.at[pl.ds(c * CH, CH)])

    @pl.when(sid == NS - 1)
    def _():
        pltpu.sync_copy(ma.at[pl.ds(0, ZTAIL)],
                        acc.at[pl.ds(NFULL * CH, ZTAIL)])

    plsc.subcore_barrier()

    def relu_add(m, a):
        def rrow(r, _):
            for cc in range(HID // 16):
                sl = pl.ds(cc * 16, 16)
                m[r, sl] = jnp.maximum(m[r, sl] + a[r, sl], 0.0)
            return 0

        lax.fori_loop(0, CH, rrow, 0)

    def block(bj, _):
        pltpu.sync_copy(src_hbm.at[sid, bj], si)
        pltpu.sync_copy(rc_hbm.at[sid, bj], rci)

        @pl.when(cid == 0)
        def _():
            pltpu.sync_copy(dst0_hbm.at[sid, bj], di)

        @pl.when(cid == 1)
        def _():
            pltpu.sync_copy(dst1_hbm.at[sid, bj], di)

        pltpu.async_copy(h_hbm.at[si.at[0]], ma, sha)
        pltpu.async_copy(pair_hbm.at[rci.at[0]], aa, spa)
        pltpu.async_copy(h_hbm.at[si.at[1]], mb, shb)
        pltpu.async_copy(pair_hbm.at[rci.at[1]], ab, spb)

        def pair(k, _):
            j0 = 2 * k
            j1 = j0 + 1
            pltpu.make_async_copy(h_hbm.at[si.at[j0]], ma, sha).wait()
            pltpu.make_async_copy(pair_hbm.at[rci.at[j0]], aa, spa).wait()
            relu_add(ma, aa)
            pltpu.sync_copy(ma, acc.at[di.at[j0]], add=True)

            @pl.when(k < NPAIR - 1)
            def _():
                pltpu.async_copy(h_hbm.at[si.at[j0 + 2]], ma, sha)
                pltpu.async_copy(pair_hbm.at[rci.at[j0 + 2]], aa, spa)

            pltpu.make_async_copy(h_hbm.at[si.at[j1]], mb, shb).wait()
            pltpu.make_async_copy(pair_hbm.at[rci.at[j1]], ab, spb).wait()
            relu_add(mb, ab)
            pltpu.sync_copy(mb, acc.at[di.at[j1]], add=True)

            @pl.when(k < NPAIR - 1)
            def _():
                pltpu.async_copy(h_hbm.at[si.at[j1 + 2]], mb, shb)
                pltpu.async_copy(pair_hbm.at[rci.at[j1 + 2]], ab, spb)
            return 0

        lax.fori_loop(0, NPAIR, pair, 0)
        return 0

    lax.fori_loop(0, NBLK, block, 0)
    plsc.subcore_barrier()

    for k in range((NFULL + NS - 1) // NS):
        c = sid + k * NS

        @pl.when(c < NFULL)
        def _():
            pltpu.sync_copy(acc.at[pl.ds(c * CH, CH)],
                            out_hbm.at[pl.ds(cid * NHALF + c * CH, CH)])

    @pl.when(sid == NS - 1)
    def _():
        pltpu.sync_copy(acc.at[pl.ds(NFULL * CH, TAIL)],
                        out_hbm.at[pl.ds(cid * NHALF + NFULL * CH, TAIL)])


def _pair_body(role_ref, child_ref, o_ref):
    for rr in range(8):
        o_ref[pl.ds(rr * NCHILD, NCHILD), :] = (
            child_ref[...] + role_ref[pl.ds(rr, 1), :])


def _pair_table(role_emb, child_emb):
    return pl.pallas_call(
        _pair_body,
        grid=(NROLE // 8,),
        in_specs=[
            pl.BlockSpec((8, HID), lambda i: (i, 0)),
            pl.BlockSpec((NCHILD, HID), lambda i: (0, 0)),
        ],
        out_specs=pl.BlockSpec((8 * NCHILD, HID), lambda i: (i, 0)),
        out_shape=jax.ShapeDtypeStruct((NRC, HID), jnp.float32),
    )(role_emb, child_emb)


def _sem_body(base_ref, xs_ref, w_ref, b_ref, o_ref):
    o_ref[...] = (base_ref[...] +
                  jnp.dot(xs_ref[...], w_ref[...],
                          preferred_element_type=jnp.float32, precision=_PREC) +
                  b_ref[...])


def _sem_matmul(base, x_sem, W_sem, b_sem2d):
    k = x_sem.shape[1]
    return pl.pallas_call(
        _sem_body,
        grid=(NB,),
        in_specs=[
            pl.BlockSpec((MB, HID), lambda i: (i, 0)),
            pl.BlockSpec((MB, k), lambda i: (i, 0)),
            pl.BlockSpec((k, HID), lambda i: (0, 0)),
            pl.BlockSpec((1, HID), lambda i: (0, 0)),
        ],
        out_specs=pl.BlockSpec((MB, HID), lambda i: (i, 0)),
        out_shape=jax.ShapeDtypeStruct((N, HID), jnp.float32),
    )(base, x_sem, W_sem, b_sem2d)


def _mlp_body(h_ref, agg_ref, w1_ref, b1_ref, w2_ref, b2_ref, o_ref):
    z = h_ref[...] + agg_ref[...]
    t = jnp.maximum(
        jnp.dot(z, w1_ref[...], preferred_element_type=jnp.float32,
                precision=_PREC) + b1_ref[...], 0.0)
    o_ref[...] = (jnp.dot(t, w2_ref[...], preferred_element_type=jnp.float32,
                          precision=_PREC) + b2_ref[...])


def _mlp(h, agg, W1, b1_2d, W2, b2_2d):
    return pl.pallas_call(
        _mlp_body,
        grid=(NB,),
        in_specs=[
            pl.BlockSpec((MB, HID), lambda i: (i, 0)),
            pl.BlockSpec((MB, HID), lambda i: (i, 0)),
            pl.BlockSpec((HID, HID), lambda i: (0, 0)),
            pl.BlockSpec((1, HID), lambda i: (0, 0)),
            pl.BlockSpec((HID, HID), lambda i: (0, 0)),
            pl.BlockSpec((1, HID), lambda i: (0, 0)),
        ],
        out_specs=pl.BlockSpec((MB, HID), lambda i: (i, 0)),
        out_shape=jax.ShapeDtypeStruct((N, HID), jnp.float32),
    )(h, agg, W1, b1_2d, W2, b2_2d)


def _pool_body(h_ref, batch_ref, wp_ref, bp_ref, o_ref, sums, cnts):
    i = pl.program_id(0)

    @pl.when(i == 0)
    def _():
        sums[...] = jnp.zeros_like(sums)
        cnts[...] = jnp.zeros_like(cnts)

    b = batch_ref[0, 0]
    mask = (b[:, None] ==
            lax.broadcasted_iota(jnp.int32, (1, B), 1)).astype(jnp.float32)
    sums[...] += lax.dot_general(mask, h_ref[...], (((0,), (0,)), ((), ())),
                                 preferred_element_type=jnp.float32,
                                 precision=_PREC)
    ones = jnp.ones((MB, HID), jnp.float32)
    cnts[...] += lax.dot_general(mask, ones, (((0,), (0,)), ((), ())),
                                 preferred_element_type=jnp.float32,
                                 precision=_PREC)

    @pl.when(i == NB - 1)
    def _():
        pooled = sums[...] / jnp.maximum(cnts[...], 1.0)
        o_ref[...] = (jnp.dot(pooled, wp_ref[...],
                              preferred_element_type=jnp.float32,
                              precision=_PREC) + bp_ref[...])


def _pool_proj(h, batch3d, W_proj, b_proj2d):
    return pl.pallas_call(
        _pool_body,
        grid=(NB,),
        in_specs=[
            pl.BlockSpec((MB, HID), lambda i: (i, 0)),
            pl.BlockSpec((1, 1, MB), lambda i: (i, 0, 0)),
            pl.BlockSpec((HID, HID), lambda i: (0, 0)),
            pl.BlockSpec((1, HID), lambda i: (0, 0)),
        ],
        out_specs=pl.BlockSpec((B, HID), lambda i: (0, 0)),
        out_shape=jax.ShapeDtypeStruct((B, HID), jnp.float32),
        scratch_shapes=[
            pltpu.VMEM((B, HID), jnp.float32),
            pltpu.VMEM((B, HID), jnp.float32),
        ],
    )(h, batch3d, W_proj, b_proj2d)


def kernel(x, x_sem, edge_index, edge_attr, batch, label_emb, type_emb,
           W_sem, b_sem, role_emb, child_emb,
           W1_0, b1_0, W2_0, b2_0, W1_1, b1_1, W2_1, b2_1,
           W_proj, b_proj):
    xt = x.T.astype(jnp.int32)
    x0 = xt[0].reshape(NROWCH, 1, CH)
    x1 = xt[1].reshape(NROWCH, 1, CH)
    ei = edge_index.astype(jnp.int32)
    src2d = ei[0].reshape(NS, NBLK, BLK, CH)
    dst = ei[1]
    dst0 = jnp.minimum(dst, NHALF).reshape(NS, NBLK, BLK, CH)
    dst1 = jnp.where(dst >= NHALF, dst - NHALF,
                     NHALF).reshape(NS, NBLK, BLK, CH)
    ea = edge_attr.T.astype(jnp.int32)
    rc2d = (ea[0] * NCHILD +
            jnp.clip(ea[1], 0, NCHILD - 1)).reshape(NS, NBLK, BLK, CH)
    batch3d = batch.astype(jnp.int32).reshape(NB, 1, MB)

    base = _node_emb(x0, x1, label_emb, type_emb)
    pair = _pair_table(role_emb, child_emb)
    h = _sem_matmul(base, x_sem, W_sem, b_sem.reshape(1, HID))
    for (W1, b1, W2, b2) in ((W1_0, b1_0, W2_0, b2_0),
                             (W1_1, b1_1, W2_1, b2_1)):
        agg = _msg_pass(src2d, dst0, dst1, rc2d, h, pair)
        h = _mlp(h, agg, W1, b1.reshape(1, HID), W2, b2.reshape(1, HID))
    return _pool_proj(h, batch3d, W_proj, b_proj.reshape(1, HID))


# speedup vs baseline: 4.5790x; 1.0135x over previous
import functools

import jax
import jax.numpy as jnp
from jax import lax
from jax.experimental import pallas as pl
from jax.experimental.pallas import tpu as pltpu
from jax.experimental.pallas import tpu_sc as plsc

N = 10000
E = 320000
HID = 128
B = 64
NROLE = 200
NCHILD = 200

NC = 2
NS = 16
NW = NC * NS

CH = 80
EPS = E // NS
NCHUNK = EPS // CH
BLK = 50
NBLK = NCHUNK // BLK
NPAIR = BLK // 2
NROWCH = N // CH
NRC = NROLE * NCHILD

NHALF = N // NC
ACC_ROWS = 5008
NFULL = NHALF // CH
TAIL = NHALF - NFULL * CH
ZTAIL = ACC_ROWS - NFULL * CH

MB = 400
NB = N // MB

_mesh = plsc.VectorSubcoreMesh(core_axis_name="c", subcore_axis_name="s")
_PREC = lax.Precision.HIGHEST


@functools.partial(
    pl.kernel,
    out_type=jax.ShapeDtypeStruct((N, HID), jnp.float32),
    mesh=_mesh,
    scratch_types=[
        pltpu.VMEM((CH,), jnp.int32),
        pltpu.VMEM((CH,), jnp.int32),
        pltpu.VMEM((CH, HID), jnp.float32),
        pltpu.SemaphoreType.DMA,
    ],
)
def _node_emb(x0_hbm, x1_hbm, lab_hbm, typ_hbm, out_hbm, xi0, xi1, rows, sem):
    w = lax.axis_index("s") * NC + lax.axis_index("c")
    for k in range((NROWCH + NW - 1) // NW):
        c = w + k * NW

        @pl.when(c < NROWCH)
        def _():
            pltpu.sync_copy(x0_hbm.at[c, 0], xi0)
            pltpu.sync_copy(x1_hbm.at[c, 0], xi1)
            pltpu.async_copy(lab_hbm.at[xi0], rows, sem).wait()
            pltpu.async_copy(typ_hbm.at[xi1], rows, sem, add=True).wait()
            pltpu.sync_copy(rows, out_hbm.at[pl.ds(c * CH, CH)])


@functools.partial(
    pl.kernel,
    out_type=jax.ShapeDtypeStruct((N, HID), jnp.float32),
    mesh=_mesh,
    scratch_types=[
        pltpu.VMEM((BLK, CH), jnp.int32),
        pltpu.VMEM((BLK, CH), jnp.int32),
        pltpu.VMEM((BLK, CH), jnp.int32),
        pltpu.VMEM((CH, HID), jnp.float32),
        pltpu.VMEM((CH, HID), jnp.float32),
        pltpu.VMEM((CH, HID), jnp.float32),
        pltpu.VMEM((CH, HID), jnp.float32),
        pltpu.VMEM_SHARED((ACC_ROWS, HID), jnp.float32),
        pltpu.SemaphoreType.DMA,
        pltpu.SemaphoreType.DMA,
        pltpu.SemaphoreType.DMA,
        pltpu.SemaphoreType.DMA,
    ],
)
def _msg_pass(src_hbm, dst0_hbm, dst1_hbm, rc_hbm,
              h_hbm, pair_hbm,
              out_hbm, si, di, rci, ma, aa, mb, ab, acc,
              sha, spa, shb, spb):
    cid = lax.axis_index("c")
    sid = lax.axis_index("s")

    zv = jnp.zeros((16,), jnp.float32)

    def zrow(r, _):
        for cc in range(HID // 16):
            ma[r, pl.ds(cc * 16, 16)] = zv
        return 0

    lax.fori_loop(0, CH, zrow, 0)

    for k in range((NFULL + NS - 1) // NS):
        c = sid + k * NS

        @pl.when(c < NFULL)
        def _():
            pltpu.sync_copy(ma, acc.at[pl.ds(c * CH, CH)])

    @pl.when(sid == NS - 1)
    def _():
        pltpu.sync_copy(ma.at[pl.ds(0, ZTAIL)],
                        acc.at[pl.ds(NFULL * CH, ZTAIL)])

    plsc.subcore_barrier()

    def relu_add(m, a):
        def rrow(r4, _):
            r = 4 * r4
            for rr in range(4):
                for cc in range(HID // 16):
                    sl = pl.ds(cc * 16, 16)
                    m[r + rr, sl] = jnp.maximum(m[r + rr, sl] + a[r + rr, sl],
                                                0.0)
            return 0

        lax.fori_loop(0, CH // 4, rrow, 0)

    def block(bj, _):
        pltpu.sync_copy(src_hbm.at[sid, bj], si)
        pltpu.sync_copy(rc_hbm.at[sid, bj], rci)

        @pl.when(cid == 0)
        def _():
            pltpu.sync_copy(dst0_hbm.at[sid, bj], di)

        @pl.when(cid == 1)
        def _():
            pltpu.sync_copy(dst1_hbm.at[sid, bj], di)

        pltpu.async_copy(h_hbm.at[si.at[0]], ma, sha)
        pltpu.async_copy(pair_hbm.at[rci.at[0]], aa, spa)
        pltpu.async_copy(h_hbm.at[si.at[1]], mb, shb)
        pltpu.async_copy(pair_hbm.at[rci.at[1]], ab, spb)

        def pair(k, _):
            j0 = 2 * k
            j1 = j0 + 1
            pltpu.make_async_copy(h_hbm.at[si.at[j0]], ma, sha).wait()
            pltpu.make_async_copy(pair_hbm.at[rci.at[j0]], aa, spa).wait()
            relu_add(ma, aa)
            pltpu.sync_copy(ma, acc.at[di.at[j0]], add=True)

            @pl.when(k < NPAIR - 1)
            def _():
                pltpu.async_copy(h_hbm.at[si.at[j0 + 2]], ma, sha)
                pltpu.async_copy(pair_hbm.at[rci.at[j0 + 2]], aa, spa)

            pltpu.make_async_copy(h_hbm.at[si.at[j1]], mb, shb).wait()
            pltpu.make_async_copy(pair_hbm.at[rci.at[j1]], ab, spb).wait()
            relu_add(mb, ab)
            pltpu.sync_copy(mb, acc.at[di.at[j1]], add=True)

            @pl.when(k < NPAIR - 1)
            def _():
                pltpu.async_copy(h_hbm.at[si.at[j1 + 2]], mb, shb)
                pltpu.async_copy(pair_hbm.at[rci.at[j1 + 2]], ab, spb)
            return 0

        lax.fori_loop(0, NPAIR, pair, 0)
        return 0

    lax.fori_loop(0, NBLK, block, 0)
    plsc.subcore_barrier()

    for k in range((NFULL + NS - 1) // NS):
        c = sid + k * NS

        @pl.when(c < NFULL)
        def _():
            pltpu.sync_copy(acc.at[pl.ds(c * CH, CH)],
                            out_hbm.at[pl.ds(cid * NHALF + c * CH, CH)])

    @pl.when(sid == NS - 1)
    def _():
        pltpu.sync_copy(acc.at[pl.ds(NFULL * CH, TAIL)],
                        out_hbm.at[pl.ds(cid * NHALF + NFULL * CH, TAIL)])


def _pair_body(role_ref, child_ref, o_ref):
    for rr in range(8):
        o_ref[pl.ds(rr * NCHILD, NCHILD), :] = (
            child_ref[...] + role_ref[pl.ds(rr, 1), :])


def _pair_table(role_emb, child_emb):
    return pl.pallas_call(
        _pair_body,
        grid=(NROLE // 8,),
        in_specs=[
            pl.BlockSpec((8, HID), lambda i: (i, 0)),
            pl.BlockSpec((NCHILD, HID), lambda i: (0, 0)),
        ],
        out_specs=pl.BlockSpec((8 * NCHILD, HID), lambda i: (i, 0)),
        out_shape=jax.ShapeDtypeStruct((NRC, HID), jnp.float32),
    )(role_emb, child_emb)


def _sem_body(base_ref, xs_ref, w_ref, b_ref, o_ref):
    o_ref[...] = (base_ref[...] +
                  jnp.dot(xs_ref[...], w_ref[...],
                          preferred_element_type=jnp.float32, precision=_PREC) +
                  b_ref[...])


def _sem_matmul(base, x_sem, W_sem, b_sem2d):
    k = x_sem.shape[1]
    return pl.pallas_call(
        _sem_body,
        grid=(NB,),
        in_specs=[
            pl.BlockSpec((MB, HID), lambda i: (i, 0)),
            pl.BlockSpec((MB, k), lambda i: (i, 0)),
            pl.BlockSpec((k, HID), lambda i: (0, 0)),
            pl.BlockSpec((1, HID), lambda i: (0, 0)),
        ],
        out_specs=pl.BlockSpec((MB, HID), lambda i: (i, 0)),
        out_shape=jax.ShapeDtypeStruct((N, HID), jnp.float32),
    )(base, x_sem, W_sem, b_sem2d)


def _mlp_body(h_ref, agg_ref, w1_ref, b1_ref, w2_ref, b2_ref, o_ref):
    z = h_ref[...] + agg_ref[...]
    t = jnp.maximum(
        jnp.dot(z, w1_ref[...], preferred_element_type=jnp.float32,
                precision=_PREC) + b1_ref[...], 0.0)
    o_ref[...] = (jnp.dot(t, w2_ref[...], preferred_element_type=jnp.float32,
                          precision=_PREC) + b2_ref[...])


def _mlp(h, agg, W1, b1_2d, W2, b2_2d):
    return pl.pallas_call(
        _mlp_body,
        grid=(NB,),
        in_specs=[
            pl.BlockSpec((MB, HID), lambda i: (i, 0)),
            pl.BlockSpec((MB, HID), lambda i: (i, 0)),
            pl.BlockSpec((HID, HID), lambda i: (0, 0)),
            pl.BlockSpec((1, HID), lambda i: (0, 0)),
            pl.BlockSpec((HID, HID), lambda i: (0, 0)),
            pl.BlockSpec((1, HID), lambda i: (0, 0)),
        ],
        out_specs=pl.BlockSpec((MB, HID), lambda i: (i, 0)),
        out_shape=jax.ShapeDtypeStruct((N, HID), jnp.float32),
    )(h, agg, W1, b1_2d, W2, b2_2d)


def _pool_body(h_ref, batch_ref, wp_ref, bp_ref, o_ref, sums, cnts):
    i = pl.program_id(0)

    @pl.when(i == 0)
    def _():
        sums[...] = jnp.zeros_like(sums)
        cnts[...] = jnp.zeros_like(cnts)

    b = batch_ref[0, 0]
    mask = (b[:, None] ==
            lax.broadcasted_iota(jnp.int32, (1, B), 1)).astype(jnp.float32)
    sums[...] += lax.dot_general(mask, h_ref[...], (((0,), (0,)), ((), ())),
                                 preferred_element_type=jnp.float32,
                                 precision=_PREC)
    ones = jnp.ones((MB, HID), jnp.float32)
    cnts[...] += lax.dot_general(mask, ones, (((0,), (0,)), ((), ())),
                                 preferred_element_type=jnp.float32,
                                 precision=_PREC)

    @pl.when(i == NB - 1)
    def _():
        pooled = sums[...] / jnp.maximum(cnts[...], 1.0)
        o_ref[...] = (jnp.dot(pooled, wp_ref[...],
                              preferred_element_type=jnp.float32,
                              precision=_PREC) + bp_ref[...])


def _pool_proj(h, batch3d, W_proj, b_proj2d):
    return pl.pallas_call(
        _pool_body,
        grid=(NB,),
        in_specs=[
            pl.BlockSpec((MB, HID), lambda i: (i, 0)),
            pl.BlockSpec((1, 1, MB), lambda i: (i, 0, 0)),
            pl.BlockSpec((HID, HID), lambda i: (0, 0)),
            pl.BlockSpec((1, HID), lambda i: (0, 0)),
        ],
        out_specs=pl.BlockSpec((B, HID), lambda i: (0, 0)),
        out_shape=jax.ShapeDtypeStruct((B, HID), jnp.float32),
        scratch_shapes=[
            pltpu.VMEM((B, HID), jnp.float32),
            pltpu.VMEM((B, HID), jnp.float32),
        ],
    )(h, batch3d, W_proj, b_proj2d)


def kernel(x, x_sem, edge_index, edge_attr, batch, label_emb, type_emb,
           W_sem, b_sem, role_emb, child_emb,
           W1_0, b1_0, W2_0, b2_0, W1_1, b1_1, W2_1, b2_1,
           W_proj, b_proj):
    xt = x.T.astype(jnp.int32)
    x0 = xt[0].reshape(NROWCH, 1, CH)
    x1 = xt[1].reshape(NROWCH, 1, CH)
    ei = edge_index.astype(jnp.int32)
    src2d = ei[0].reshape(NS, NBLK, BLK, CH)
    dst = ei[1]
    dst0 = jnp.minimum(dst, NHALF).reshape(NS, NBLK, BLK, CH)
    dst1 = jnp.where(dst >= NHALF, dst - NHALF,
                     NHALF).reshape(NS, NBLK, BLK, CH)
    ea = edge_attr.T.astype(jnp.int32)
    rc2d = (ea[0] * NCHILD +
            jnp.clip(ea[1], 0, NCHILD - 1)).reshape(NS, NBLK, BLK, CH)
    batch3d = batch.astype(jnp.int32).reshape(NB, 1, MB)

    base = _node_emb(x0, x1, label_emb, type_emb)
    pair = _pair_table(role_emb, child_emb)
    h = _sem_matmul(base, x_sem, W_sem, b_sem.reshape(1, HID))
    for (W1, b1, W2, b2) in ((W1_0, b1_0, W2_0, b2_0),
                             (W1_1, b1_1, W2_1, b2_1)):
        agg = _msg_pass(src2d, dst0, dst1, rc2d, h, pair)
        h = _mlp(h, agg, W1, b1.reshape(1, HID), W2, b2.reshape(1, HID))
    return _pool_proj(h, batch3d, W_proj, b_proj.reshape(1, HID))
